# Initial kernel scaffold; baseline (speedup 1.0000x reference)
import jax
import jax.numpy as jnp
from jax.experimental import pallas as pl


def kernel(x, edge_index_list, W1, b1, W2, b2, bn_gamma, bn_beta, attn):
    raise NotImplementedError("write your pallas kernel here")


# speedup vs baseline: 13.5540x; 13.5540x over previous
import functools

import jax
import jax.numpy as jnp
from jax import lax
from jax.experimental import pallas as pl
from jax.experimental.pallas import tpu as pltpu
from jax.experimental.pallas import tpu_sc as plsc

EPS = 1e-5

NC = 2
NS = 16
NW = NC * NS
CH = 128

TN = 2000


def _sc_deg_body(V, CPW, didx_hbm, zrow_hbm, out_hbm, didx_v, ones_v, row_v, deg_sh):
    c = lax.axis_index("c")
    s = lax.axis_index("s")
    wid = s * NC + c
    ZR = zrow_hbm.shape[0]
    NPAD = ZR * NS
    for i in range(CH // 16):
        ones_v[pl.ds(i * 16, 16)] = jnp.full((16,), 1.0, jnp.float32)
    pltpu.sync_copy(zrow_hbm, row_v)
    for v in range(V):
        pltpu.sync_copy(row_v, deg_sh.at[pl.ds(v * NPAD + s * ZR, ZR)])
    plsc.subcore_barrier()
    for v in range(V):
        def body(j, _):
            off = ((v * NW + wid) * CPW + j) * CH
            pltpu.sync_copy(didx_hbm.at[pl.ds(off, CH)], didx_v)
            pltpu.sync_copy(ones_v, deg_sh.at[didx_v], add=True)
            return ()
        lax.fori_loop(0, CPW, body, ())
    plsc.subcore_barrier()
    for v in range(V):
        pltpu.sync_copy(deg_sh.at[pl.ds(v * NPAD + s * ZR, ZR)], row_v)
        pltpu.sync_copy(row_v, out_hbm.at[pl.ds((c * V + v) * NPAD + s * ZR, ZR)])


def _sc_agg_body(V, CPW, g_hbm, sidx_hbm, didx_hbm, zblk_hbm, out_hbm,
                 sidx_v, didx_v, rows_v, zblk_v, acc_sh, sem):
    c = lax.axis_index("c")
    s = lax.axis_index("s")
    wid = s * NC + c
    NPAD = acc_sh.shape[0]
    ZR = NPAD // NS
    pltpu.sync_copy(zblk_hbm, zblk_v)
    for v in range(V):
        for i in range(ZR // CH):
            pltpu.sync_copy(zblk_v, acc_sh.at[pl.ds(s * ZR + i * CH, CH)])
        plsc.subcore_barrier()
        def body(j, _):
            off = ((v * NW + wid) * CPW + j) * CH
            pltpu.sync_copy(sidx_hbm.at[pl.ds(off, CH)], sidx_v)
            pltpu.sync_copy(didx_hbm.at[pl.ds(off, CH)], didx_v)
            pltpu.async_copy(g_hbm.at[sidx_v], rows_v, sem).wait()
            pltpu.sync_copy(rows_v, acc_sh.at[didx_v], add=True)
            return ()
        lax.fori_loop(0, CPW, body, ())
        plsc.subcore_barrier()
        for i in range(ZR // CH):
            pltpu.sync_copy(acc_sh.at[pl.ds(s * ZR + i * CH, CH)], rows_v)
            pltpu.sync_copy(rows_v, out_hbm.at[v, c, pl.ds(s * ZR + i * CH, CH)])


def _tc_stage1(x_ref, w_ref, degp_ref, g_out, dinv_out):
    deg = degp_ref[0, 0] + degp_ref[1, 0] + 1.0
    dinv = lax.rsqrt(deg)
    h = jnp.dot(x_ref[...], w_ref[0], preferred_element_type=jnp.float32)
    g_out[0] = h * dinv
    dinv_out[0] = dinv


def _tc_stage2(N, acc_ref, g_ref, dinv_ref, bng_ref, bnb_ref, w2_ref, g2_out,
               ssum_ref, ssq_ref):
    p = pl.program_id(1)
    t = pl.program_id(2)
    D = g_ref.shape[-1]
    z = (acc_ref[0, 0] + acc_ref[0, 1] + g_ref[0]) * dinv_ref[0]

    @pl.when(p == 0)
    def _():
        @pl.when(t == 0)
        def _():
            ssum_ref[...] = jnp.zeros((8, D), jnp.float32)
            ssq_ref[...] = jnp.zeros((8, D), jnp.float32)
        ssum_ref[0:1, :] += jnp.sum(z, axis=0, keepdims=True)
        ssq_ref[0:1, :] += jnp.sum(z * z, axis=0, keepdims=True)

    @pl.when(p == 1)
    def _():
        mu = ssum_ref[0:1, :] * (1.0 / N)
        m2 = ssq_ref[0:1, :] * (1.0 / N)
        rs = lax.rsqrt(jnp.maximum(m2 - mu * mu, 0.0) + EPS)
        a_row = bng_ref[0:1, :] * rs
        b_row = bnb_ref[0:1, :] - a_row * mu
        r = jnp.maximum(a_row * z + b_row, 0.0)
        g2_out[0] = jnp.dot(r, w2_ref[0], preferred_element_type=jnp.float32) \
            * dinv_ref[0]


def _tc_stage3(N, V, acc_ref, g_ref, dinv_ref, bng_ref, bnb_ref, aw_ref,
               out_ref, ssum_ref, ssq_ref):
    p = pl.program_id(0)
    t = pl.program_id(1)
    v = pl.program_id(2)
    D = g_ref.shape[-1]
    z = (acc_ref[0, 0] + acc_ref[0, 1] + g_ref[0]) * dinv_ref[0]

    for k in range(V):
        @pl.when((p == 0) & (v == k))
        def _():
            @pl.when(t == 0)
            def _():
                ssum_ref[k] = jnp.zeros((8, D), jnp.float32)
                ssq_ref[k] = jnp.zeros((8, D), jnp.float32)
            ssum_ref[k, 0:1, :] += jnp.sum(z, axis=0, keepdims=True)
            ssq_ref[k, 0:1, :] += jnp.sum(z * z, axis=0, keepdims=True)

        @pl.when((p == 1) & (v == k))
        def _():
            mu = ssum_ref[k, 0:1, :] * (1.0 / N)
            m2 = ssq_ref[k, 0:1, :] * (1.0 / N)
            rs = lax.rsqrt(jnp.maximum(m2 - mu * mu, 0.0) + EPS)
            aw = aw_ref[k, 0]
            a_row = aw * bng_ref[1:2, :] * rs
            b_row = aw * bnb_ref[1:2, :] - a_row * mu
            contrib = a_row * z + b_row
            if k == 0:
                out_ref[...] = contrib
            else:
                out_ref[...] += contrib


def kernel(x, edge_index_list, W1, b1, W2, b2, bn_gamma, bn_beta, attn):
    del b1, b2
    N, D = x.shape
    V, _, E = edge_index_list.shape
    f32 = jnp.float32
    NT = N // TN

    CPW = -(-E // (NW * CH))
    E_pad = NW * CPW * CH
    NPAD = -(-N // (NS * CH)) * (NS * CH)
    ZR = NPAD // NS

    pad_n = E_pad - E
    src = edge_index_list[:, 0, :]
    dst = edge_index_list[:, 1, :]
    pad_src = (jnp.arange(pad_n, dtype=jnp.int32) % N)[None, :] + jnp.zeros((V, 1), jnp.int32)
    pad_dst = (N + jnp.arange(pad_n, dtype=jnp.int32) % (NPAD - N))[None, :] + jnp.zeros((V, 1), jnp.int32)
    src_p = jnp.concatenate([src, pad_src], axis=1)
    dst_p = jnp.concatenate([dst, pad_dst], axis=1)
    voff = (jnp.arange(V, dtype=jnp.int32) * N)[:, None]
    voff_pad = (jnp.arange(V, dtype=jnp.int32) * NPAD)[:, None]
    sidx_g = (src_p + voff).reshape(V * E_pad)
    didx_r = dst_p.reshape(V * E_pad)
    didx_deg = (dst_p + voff_pad).reshape(V * E_pad)

    zrow = jnp.zeros((ZR,), f32)
    zblk = jnp.zeros((CH, D), f32)
    aw = jax.nn.softmax(attn).reshape(V, 1).astype(f32)

    mesh = plsc.VectorSubcoreMesh(core_axis_name="c", subcore_axis_name="s")

    deg_call = pl.kernel(
        functools.partial(_sc_deg_body, V, CPW),
        out_type=jax.ShapeDtypeStruct((NC * V * NPAD,), f32),
        mesh=mesh,
        scratch_types=[
            pltpu.VMEM((CH,), jnp.int32),
            pltpu.VMEM((CH,), f32),
            pltpu.VMEM((ZR,), f32),
            pltpu.VMEM_SHARED((V * NPAD,), f32),
        ],
    )
    degp = deg_call(didx_deg, zrow)
    degp4 = degp.reshape(NC, V, NPAD, 1)

    agg_call = pl.kernel(
        functools.partial(_sc_agg_body, V, CPW),
        out_type=jax.ShapeDtypeStruct((V, NC, NPAD, D), f32),
        mesh=mesh,
        scratch_types=[
            pltpu.VMEM((CH,), jnp.int32),
            pltpu.VMEM((CH,), jnp.int32),
            pltpu.VMEM((CH, D), f32),
            pltpu.VMEM((CH, D), f32),
            pltpu.VMEM_SHARED((NPAD, D), f32),
            pltpu.SemaphoreType.DMA,
        ],
    )

    stage1 = pl.pallas_call(
        _tc_stage1,
        grid=(NT, V),
        in_specs=[
            pl.BlockSpec((TN, D), lambda t, v: (t, 0)),
            pl.BlockSpec((1, D, D), lambda t, v: (v, 0, 0)),
            pl.BlockSpec((NC, 1, TN, 1), lambda t, v: (0, v, t, 0)),
        ],
        out_specs=[
            pl.BlockSpec((1, TN, D), lambda t, v: (v, t, 0)),
            pl.BlockSpec((1, TN, 1), lambda t, v: (v, t, 0)),
        ],
        out_shape=[
            jax.ShapeDtypeStruct((V, N, D), f32),
            jax.ShapeDtypeStruct((V, N, 1), f32),
        ],
    )
    g1, dinv = stage1(x, W1, degp4)

    acc1 = agg_call(g1.reshape(V * N, D), sidx_g, didx_r, zblk)

    stage2 = pl.pallas_call(
        functools.partial(_tc_stage2, N),
        grid=(V, 2, NT),
        in_specs=[
            pl.BlockSpec((1, NC, TN, D), lambda v, p, t: (v, 0, t, 0)),
            pl.BlockSpec((1, TN, D), lambda v, p, t: (v, t, 0)),
            pl.BlockSpec((1, TN, 1), lambda v, p, t: (v, t, 0)),
            pl.BlockSpec((2, D), lambda v, p, t: (0, 0)),
            pl.BlockSpec((2, D), lambda v, p, t: (0, 0)),
            pl.BlockSpec((1, D, D), lambda v, p, t: (v, 0, 0)),
        ],
        out_specs=pl.BlockSpec((1, TN, D), lambda v, p, t: (v, t, 0)),
        out_shape=jax.ShapeDtypeStruct((V, N, D), f32),
        scratch_shapes=[
            pltpu.VMEM((8, D), f32),
            pltpu.VMEM((8, D), f32),
        ],
    )
    g2 = stage2(acc1, g1, dinv, bn_gamma, bn_beta, W2)

    acc2 = agg_call(g2.reshape(V * N, D), sidx_g, didx_r, zblk)

    stage3 = pl.pallas_call(
        functools.partial(_tc_stage3, N, V),
        grid=(2, NT, V),
        in_specs=[
            pl.BlockSpec((1, NC, TN, D), lambda p, t, v: (v, 0, t, 0)),
            pl.BlockSpec((1, TN, D), lambda p, t, v: (v, t, 0)),
            pl.BlockSpec((1, TN, 1), lambda p, t, v: (v, t, 0)),
            pl.BlockSpec((2, D), lambda p, t, v: (0, 0)),
            pl.BlockSpec((2, D), lambda p, t, v: (0, 0)),
            pl.BlockSpec(memory_space=pltpu.SMEM),
        ],
        out_specs=pl.BlockSpec((TN, D), lambda p, t, v: (t, 0)),
        out_shape=jax.ShapeDtypeStruct((N, D), f32),
        scratch_shapes=[
            pltpu.VMEM((V, 8, D), f32),
            pltpu.VMEM((V, 8, D), f32),
        ],
    )
    return stage3(acc2, g2, dinv, bn_gamma, bn_beta, aw)


# speedup vs baseline: 27.5270x; 2.0309x over previous
import functools

import jax
import jax.numpy as jnp
from jax import lax
from jax.experimental import pallas as pl
from jax.experimental.pallas import tpu as pltpu
from jax.experimental.pallas import tpu_sc as plsc

EPS = 1e-5

NC = 2
NS = 16
NW = NC * NS
CH = 128

TN = 2000


NBUF = 2
NG = 4


def _sc_deg_body(V, CPW, didx_hbm, zrow_hbm, out_hbm, didx_v, ones_v, row_v, deg_sh):
    c = lax.axis_index("c")
    s = lax.axis_index("s")
    wid = s * NC + c
    ZR = zrow_hbm.shape[0]
    NPAD = ZR * NS
    for i in range(CH // 16):
        ones_v[pl.ds(i * 16, 16)] = jnp.full((16,), 1.0, jnp.float32)
    pltpu.sync_copy(zrow_hbm, row_v)
    for v in range(V):
        pltpu.sync_copy(row_v, deg_sh.at[pl.ds(v * NPAD + s * ZR, ZR)])
    plsc.subcore_barrier()
    for v in range(V):
        pltpu.sync_copy(didx_hbm.at[v, wid], didx_v)

        def body(j, _):
            pltpu.sync_copy(ones_v, deg_sh.at[didx_v.at[j]], add=True)
            return ()
        lax.fori_loop(0, CPW, body, ())
    plsc.subcore_barrier()
    for v in range(V):
        pltpu.sync_copy(deg_sh.at[pl.ds(v * NPAD + s * ZR, ZR)], row_v)
        pltpu.sync_copy(row_v, out_hbm.at[pl.ds((c * V + v) * NPAD + s * ZR, ZR)])


def _sc_agg_body(V, G, g_hbm, sidx_hbm, didx_hbm, zblk_hbm, out_hbm,
                 si0, di0, si1, di1, r0, r1, acc_sh, g0, g1):
    c = lax.axis_index("c")
    s = lax.axis_index("s")
    wid = s * NC + c
    NPAD = acc_sh.shape[0]
    ZR = NPAD // NS
    rows = [r0, r1]
    gsem = [g0, g1]
    sibuf = [si0, si1]
    dibuf = [di0, di1]
    drain_src = g_hbm.at[pl.ds(0, CH)]
    for v in range(V):
        pltpu.sync_copy(zblk_hbm, r0)
        for i in range(ZR // CH):
            pltpu.sync_copy(r0, acc_sh.at[pl.ds(s * ZR + i * CH, CH)])
        pltpu.sync_copy(sidx_hbm.at[v, wid, 0], si0)
        pltpu.sync_copy(didx_hbm.at[v, wid, 0], di0)
        plsc.subcore_barrier()
        for b in range(NBUF):
            pltpu.async_copy(g_hbm.at[si0.at[b]], rows[b], gsem[b])
        for gi in range(NG):
            pb, nb = gi % 2, (gi + 1) % 2
            if gi + 1 < NG:
                pltpu.sync_copy(sidx_hbm.at[v, wid, gi + 1], sibuf[nb])
                pltpu.sync_copy(didx_hbm.at[v, wid, gi + 1], dibuf[nb])

            def body(t, _):
                for b in range(NBUF):
                    jrel = t * NBUF + b
                    pltpu.make_async_copy(drain_src, rows[b], gsem[b]).wait()
                    pltpu.sync_copy(rows[b], acc_sh.at[dibuf[pb].at[jrel]],
                                    add=True)
                    pltpu.async_copy(g_hbm.at[sibuf[pb].at[jrel + NBUF]],
                                     rows[b], gsem[b])
                return ()
            lax.fori_loop(0, (G - NBUF) // NBUF, body, ())
            for b in range(NBUF):
                jrel = G - NBUF + b
                pltpu.make_async_copy(drain_src, rows[b], gsem[b]).wait()
                pltpu.sync_copy(rows[b], acc_sh.at[dibuf[pb].at[jrel]],
                                add=True)
                if gi + 1 < NG:
                    pltpu.async_copy(g_hbm.at[sibuf[nb].at[b]], rows[b],
                                     gsem[b])
        plsc.subcore_barrier()
        for i in range(ZR // CH):
            pltpu.sync_copy(acc_sh.at[pl.ds(s * ZR + i * CH, CH)], r0)
            pltpu.sync_copy(r0, out_hbm.at[v, c, pl.ds(s * ZR + i * CH, CH)])


def _tc_stage1(x_ref, w_ref, degp_ref, g_out, dinv_out):
    deg = degp_ref[0, 0] + degp_ref[1, 0] + 1.0
    dinv = lax.rsqrt(deg)
    h = jnp.dot(x_ref[...], w_ref[0], preferred_element_type=jnp.float32)
    g_out[0] = h * dinv
    dinv_out[0] = dinv


def _tc_stage2(N, acc_ref, g_ref, dinv_ref, bng_ref, bnb_ref, w2_ref, g2_out,
               ssum_ref, ssq_ref):
    p = pl.program_id(1)
    t = pl.program_id(2)
    D = g_ref.shape[-1]
    z = (acc_ref[0, 0] + acc_ref[0, 1] + g_ref[0]) * dinv_ref[0]

    @pl.when(p == 0)
    def _():
        @pl.when(t == 0)
        def _():
            ssum_ref[...] = jnp.zeros((8, D), jnp.float32)
            ssq_ref[...] = jnp.zeros((8, D), jnp.float32)
        ssum_ref[0:1, :] += jnp.sum(z, axis=0, keepdims=True)
        ssq_ref[0:1, :] += jnp.sum(z * z, axis=0, keepdims=True)

    @pl.when(p == 1)
    def _():
        mu = ssum_ref[0:1, :] * (1.0 / N)
        m2 = ssq_ref[0:1, :] * (1.0 / N)
        rs = lax.rsqrt(jnp.maximum(m2 - mu * mu, 0.0) + EPS)
        a_row = bng_ref[0:1, :] * rs
        b_row = bnb_ref[0:1, :] - a_row * mu
        r = jnp.maximum(a_row * z + b_row, 0.0)
        g2_out[0] = jnp.dot(r, w2_ref[0], preferred_element_type=jnp.float32) \
            * dinv_ref[0]


def _tc_stage3(N, V, acc_ref, g_ref, dinv_ref, bng_ref, bnb_ref, aw_ref,
               out_ref, ssum_ref, ssq_ref):
    p = pl.program_id(0)
    t = pl.program_id(1)
    v = pl.program_id(2)
    D = g_ref.shape[-1]
    z = (acc_ref[0, 0] + acc_ref[0, 1] + g_ref[0]) * dinv_ref[0]

    for k in range(V):
        @pl.when((p == 0) & (v == k))
        def _():
            @pl.when(t == 0)
            def _():
                ssum_ref[k] = jnp.zeros((8, D), jnp.float32)
                ssq_ref[k] = jnp.zeros((8, D), jnp.float32)
            ssum_ref[k, 0:1, :] += jnp.sum(z, axis=0, keepdims=True)
            ssq_ref[k, 0:1, :] += jnp.sum(z * z, axis=0, keepdims=True)

        @pl.when((p == 1) & (v == k))
        def _():
            mu = ssum_ref[k, 0:1, :] * (1.0 / N)
            m2 = ssq_ref[k, 0:1, :] * (1.0 / N)
            rs = lax.rsqrt(jnp.maximum(m2 - mu * mu, 0.0) + EPS)
            aw = aw_ref[k, 0]
            a_row = aw * bng_ref[1:2, :] * rs
            b_row = aw * bnb_ref[1:2, :] - a_row * mu
            contrib = a_row * z + b_row
            if k == 0:
                out_ref[...] = contrib
            else:
                out_ref[...] += contrib


def kernel(x, edge_index_list, W1, b1, W2, b2, bn_gamma, bn_beta, attn):
    del b1, b2
    N, D = x.shape
    V, _, E = edge_index_list.shape
    f32 = jnp.float32
    NT = N // TN

    CPW = NG * NBUF * (-(-E // (NW * CH * NG * NBUF)))
    G = CPW // NG
    E_pad = NW * CPW * CH
    NPAD = -(-N // (NS * CH)) * (NS * CH)
    ZR = NPAD // NS

    pad_n = E_pad - E
    src = edge_index_list[:, 0, :]
    dst = edge_index_list[:, 1, :]
    pad_src = (jnp.arange(pad_n, dtype=jnp.int32) % N)[None, :] + jnp.zeros((V, 1), jnp.int32)
    pad_dst = (N + jnp.arange(pad_n, dtype=jnp.int32) % (NPAD - N))[None, :] + jnp.zeros((V, 1), jnp.int32)
    src_p = jnp.concatenate([src, pad_src], axis=1)
    dst_p = jnp.concatenate([dst, pad_dst], axis=1)
    voff = (jnp.arange(V, dtype=jnp.int32) * N)[:, None]
    voff_pad = (jnp.arange(V, dtype=jnp.int32) * NPAD)[:, None]
    sidx_g = (src_p + voff).reshape(V, NW, NG, G, CH)
    didx_r = dst_p.reshape(V, NW, NG, G, CH)
    didx_deg = (dst_p + voff_pad).reshape(V, NW, CPW, CH)

    zrow = jnp.zeros((ZR,), f32)
    zblk = jnp.zeros((CH, D), f32)
    aw = jax.nn.softmax(attn).reshape(V, 1).astype(f32)

    mesh = plsc.VectorSubcoreMesh(core_axis_name="c", subcore_axis_name="s")

    deg_call = pl.kernel(
        functools.partial(_sc_deg_body, V, CPW),
        out_type=jax.ShapeDtypeStruct((NC * V * NPAD,), f32),
        mesh=mesh,
        scratch_types=[
            pltpu.VMEM((CPW, CH), jnp.int32),
            pltpu.VMEM((CH,), f32),
            pltpu.VMEM((ZR,), f32),
            pltpu.VMEM_SHARED((V * NPAD,), f32),
        ],
    )
    degp = deg_call(didx_deg, zrow)
    degp4 = degp.reshape(NC, V, NPAD, 1)

    agg_call = pl.kernel(
        functools.partial(_sc_agg_body, V, G),
        out_type=jax.ShapeDtypeStruct((V, NC, NPAD, D), f32),
        mesh=mesh,
        scratch_types=[
            pltpu.VMEM((G, CH), jnp.int32),
            pltpu.VMEM((G, CH), jnp.int32),
            pltpu.VMEM((G, CH), jnp.int32),
            pltpu.VMEM((G, CH), jnp.int32),
            pltpu.VMEM((CH, D), f32),
            pltpu.VMEM((CH, D), f32),
            pltpu.VMEM_SHARED((NPAD, D), f32),
            pltpu.SemaphoreType.DMA,
            pltpu.SemaphoreType.DMA,
        ],
    )

    stage1 = pl.pallas_call(
        _tc_stage1,
        grid=(NT, V),
        in_specs=[
            pl.BlockSpec((TN, D), lambda t, v: (t, 0)),
            pl.BlockSpec((1, D, D), lambda t, v: (v, 0, 0)),
            pl.BlockSpec((NC, 1, TN, 1), lambda t, v: (0, v, t, 0)),
        ],
        out_specs=[
            pl.BlockSpec((1, TN, D), lambda t, v: (v, t, 0)),
            pl.BlockSpec((1, TN, 1), lambda t, v: (v, t, 0)),
        ],
        out_shape=[
            jax.ShapeDtypeStruct((V, N, D), f32),
            jax.ShapeDtypeStruct((V, N, 1), f32),
        ],
    )
    g1, dinv = stage1(x, W1, degp4)

    acc1 = agg_call(g1.reshape(V * N, D), sidx_g, didx_r, zblk)

    stage2 = pl.pallas_call(
        functools.partial(_tc_stage2, N),
        grid=(V, 2, NT),
        in_specs=[
            pl.BlockSpec((1, NC, TN, D), lambda v, p, t: (v, 0, t, 0)),
            pl.BlockSpec((1, TN, D), lambda v, p, t: (v, t, 0)),
            pl.BlockSpec((1, TN, 1), lambda v, p, t: (v, t, 0)),
            pl.BlockSpec((2, D), lambda v, p, t: (0, 0)),
            pl.BlockSpec((2, D), lambda v, p, t: (0, 0)),
            pl.BlockSpec((1, D, D), lambda v, p, t: (v, 0, 0)),
        ],
        out_specs=pl.BlockSpec((1, TN, D), lambda v, p, t: (v, t, 0)),
        out_shape=jax.ShapeDtypeStruct((V, N, D), f32),
        scratch_shapes=[
            pltpu.VMEM((8, D), f32),
            pltpu.VMEM((8, D), f32),
        ],
    )
    g2 = stage2(acc1, g1, dinv, bn_gamma, bn_beta, W2)

    acc2 = agg_call(g2.reshape(V * N, D), sidx_g, didx_r, zblk)

    stage3 = pl.pallas_call(
        functools.partial(_tc_stage3, N, V),
        grid=(2, NT, V),
        in_specs=[
            pl.BlockSpec((1, NC, TN, D), lambda p, t, v: (v, 0, t, 0)),
            pl.BlockSpec((1, TN, D), lambda p, t, v: (v, t, 0)),
            pl.BlockSpec((1, TN, 1), lambda p, t, v: (v, t, 0)),
            pl.BlockSpec((2, D), lambda p, t, v: (0, 0)),
            pl.BlockSpec((2, D), lambda p, t, v: (0, 0)),
            pl.BlockSpec(memory_space=pltpu.SMEM),
        ],
        out_specs=pl.BlockSpec((TN, D), lambda p, t, v: (t, 0)),
        out_shape=jax.ShapeDtypeStruct((N, D), f32),
        scratch_shapes=[
            pltpu.VMEM((V, 8, D), f32),
            pltpu.VMEM((V, 8, D), f32),
        ],
    )
    return stage3(acc2, g2, dinv, bn_gamma, bn_beta, aw)


# speedup vs baseline: 28.0939x; 1.0206x over previous
import functools

import jax
import jax.numpy as jnp
from jax import lax
from jax.experimental import pallas as pl
from jax.experimental.pallas import tpu as pltpu
from jax.experimental.pallas import tpu_sc as plsc

EPS = 1e-5

NC = 2
NS = 16
NW = NC * NS
CH = 128

TN = 2000


NBUF = 3
NG = 6
CHG = 96
WZ = 64


def _sc_deg_body(V, CPW, didx_hbm, zrow_hbm, out_hbm, didx_v, ones_v, row_v, deg_sh):
    c = lax.axis_index("c")
    s = lax.axis_index("s")
    wid = s * NC + c
    ZR = zrow_hbm.shape[0]
    NPAD = ZR * NS
    for i in range(CH // 16):
        ones_v[pl.ds(i * 16, 16)] = jnp.full((16,), 1.0, jnp.float32)
    pltpu.sync_copy(zrow_hbm, row_v)
    for v in range(V):
        pltpu.sync_copy(row_v, deg_sh.at[pl.ds(v * NPAD + s * ZR, ZR)])
    plsc.subcore_barrier()
    for v in range(V):
        pltpu.sync_copy(didx_hbm.at[v, wid], didx_v)

        def body(j, _):
            pltpu.sync_copy(ones_v, deg_sh.at[didx_v.at[j]], add=True)
            return ()
        lax.fori_loop(0, CPW, body, ())
    plsc.subcore_barrier()
    for v in range(V):
        pltpu.sync_copy(deg_sh.at[pl.ds(v * NPAD + s * ZR, ZR)], row_v)
        pltpu.sync_copy(row_v, out_hbm.at[pl.ds((c * V + v) * NPAD + s * ZR, ZR)])


def _sc_agg_body(V, G, g_hbm, sidx_hbm, didx_hbm, zblk_hbm, out_hbm,
                 si0, di0, si1, di1, r0, r1, r2, acc_sh,
                 g0, g1, g2, s0, s1, s2):
    c = lax.axis_index("c")
    s = lax.axis_index("s")
    wid = s * NC + c
    NPAD = acc_sh.shape[0]
    ZR = NPAD // NS
    rows = [r0, r1, r2]
    gsem = [g0, g1, g2]
    ssem = [s0, s1, s2]
    sibuf = [si0, si1]
    dibuf = [di0, di1]
    gdrain = g_hbm.at[pl.ds(0, CHG)]

    for v in range(V):
        pltpu.sync_copy(zblk_hbm, r0.at[pl.ds(0, WZ)])
        for i in range(ZR // WZ):
            pltpu.sync_copy(r0.at[pl.ds(0, WZ)],
                            acc_sh.at[pl.ds(s * ZR + i * WZ, WZ)])
        pltpu.sync_copy(sidx_hbm.at[v, wid, 0], si0)
        pltpu.sync_copy(didx_hbm.at[v, wid, 0], di0)
        plsc.subcore_barrier()

        def chunk(pb, nb, jrel, slot, fire_rel, fire_in_group, swait):
            pltpu.make_async_copy(gdrain, rows[slot], gsem[slot]).wait()
            pltpu.async_copy(rows[slot], acc_sh.at[dibuf[pb].at[jrel]],
                             ssem[slot], add=True)
            if fire_rel is not None:
                fslot = (slot + 2) % NBUF
                if swait:
                    pltpu.make_async_copy(gdrain, rows[fslot],
                                          ssem[fslot]).wait()
                src = sibuf[pb] if fire_in_group else sibuf[nb]
                pltpu.async_copy(g_hbm.at[src.at[fire_rel]], rows[fslot],
                                 gsem[fslot])

        pltpu.async_copy(g_hbm.at[si0.at[0]], rows[0], gsem[0])
        pltpu.async_copy(g_hbm.at[si0.at[1]], rows[1], gsem[1])
        TR = G // NBUF
        for gi in range(NG):
            pb, nb = gi % 2, (gi + 1) % 2
            if gi + 1 < NG:
                pltpu.sync_copy(sidx_hbm.at[v, wid, gi + 1], sibuf[nb])
                pltpu.sync_copy(didx_hbm.at[v, wid, gi + 1], dibuf[nb])
            if gi == 0:
                chunk(pb, nb, 0, 0, 2, True, False)
                chunk(pb, nb, 1, 1, 3, True, True)
                chunk(pb, nb, 2, 2, 4, True, True)
                t_lo = 1
            else:
                t_lo = 0

            def body(t, _):
                for b in range(NBUF):
                    chunk(pb, nb, t * NBUF + b, b, t * NBUF + b + 2, True,
                          True)
                return ()
            lax.fori_loop(t_lo, TR - 1, body, ())
            base = G - NBUF
            last = gi + 1 >= NG
            chunk(pb, nb, base, 0, base + 2, True, True)
            chunk(pb, nb, base + 1, 1, None if last else 0, False, True)
            chunk(pb, nb, base + 2, 2, None if last else 1, False, True)
        for b in range(NBUF):
            pltpu.make_async_copy(gdrain, rows[b], ssem[b]).wait()
        plsc.subcore_barrier()
        for i in range(ZR // WZ):
            pltpu.sync_copy(acc_sh.at[pl.ds(s * ZR + i * WZ, WZ)],
                            r0.at[pl.ds(0, WZ)])
            pltpu.sync_copy(r0.at[pl.ds(0, WZ)],
                            out_hbm.at[v, c, pl.ds(s * ZR + i * WZ, WZ)])


def _tc_stage1(x_ref, w_ref, degp_ref, g_out, dinv_out):
    deg = degp_ref[0, 0] + degp_ref[1, 0] + 1.0
    dinv = lax.rsqrt(deg)
    h = jnp.dot(x_ref[...], w_ref[0], preferred_element_type=jnp.float32)
    g_out[0] = h * dinv
    dinv_out[0] = dinv


def _tc_stage2(N, acc_ref, g_ref, dinv_ref, bng_ref, bnb_ref, w2_ref, g2_out,
               ssum_ref, ssq_ref):
    p = pl.program_id(1)
    t = pl.program_id(2)
    D = g_ref.shape[-1]
    z = (acc_ref[0, 0] + acc_ref[0, 1] + g_ref[0]) * dinv_ref[0]

    @pl.when(p == 0)
    def _():
        @pl.when(t == 0)
        def _():
            ssum_ref[...] = jnp.zeros((8, D), jnp.float32)
            ssq_ref[...] = jnp.zeros((8, D), jnp.float32)
        ssum_ref[0:1, :] += jnp.sum(z, axis=0, keepdims=True)
        ssq_ref[0:1, :] += jnp.sum(z * z, axis=0, keepdims=True)

    @pl.when(p == 1)
    def _():
        mu = ssum_ref[0:1, :] * (1.0 / N)
        m2 = ssq_ref[0:1, :] * (1.0 / N)
        rs = lax.rsqrt(jnp.maximum(m2 - mu * mu, 0.0) + EPS)
        a_row = bng_ref[0:1, :] * rs
        b_row = bnb_ref[0:1, :] - a_row * mu
        r = jnp.maximum(a_row * z + b_row, 0.0)
        g2_out[0] = jnp.dot(r, w2_ref[0], preferred_element_type=jnp.float32) \
            * dinv_ref[0]


def _tc_stage3(N, V, acc_ref, g_ref, dinv_ref, bng_ref, bnb_ref, aw_ref,
               out_ref, ssum_ref, ssq_ref):
    p = pl.program_id(0)
    t = pl.program_id(1)
    v = pl.program_id(2)
    D = g_ref.shape[-1]
    z = (acc_ref[0, 0] + acc_ref[0, 1] + g_ref[0]) * dinv_ref[0]

    for k in range(V):
        @pl.when((p == 0) & (v == k))
        def _():
            @pl.when(t == 0)
            def _():
                ssum_ref[k] = jnp.zeros((8, D), jnp.float32)
                ssq_ref[k] = jnp.zeros((8, D), jnp.float32)
            ssum_ref[k, 0:1, :] += jnp.sum(z, axis=0, keepdims=True)
            ssq_ref[k, 0:1, :] += jnp.sum(z * z, axis=0, keepdims=True)

        @pl.when((p == 1) & (v == k))
        def _():
            mu = ssum_ref[k, 0:1, :] * (1.0 / N)
            m2 = ssq_ref[k, 0:1, :] * (1.0 / N)
            rs = lax.rsqrt(jnp.maximum(m2 - mu * mu, 0.0) + EPS)
            aw = aw_ref[k, 0]
            a_row = aw * bng_ref[1:2, :] * rs
            b_row = aw * bnb_ref[1:2, :] - a_row * mu
            contrib = a_row * z + b_row
            if k == 0:
                out_ref[...] = contrib
            else:
                out_ref[...] += contrib


def kernel(x, edge_index_list, W1, b1, W2, b2, bn_gamma, bn_beta, attn):
    del b1, b2
    N, D = x.shape
    V, _, E = edge_index_list.shape
    f32 = jnp.float32
    NT = N // TN

    G = NBUF * (-(-E // (NW * CHG * NG * NBUF)))
    CPW = NG * G
    E_pad = NW * CPW * CHG
    CPWD = E_pad // (NW * CH)
    NPAD = -(-N // (NS * CH)) * (NS * CH)
    ZR = NPAD // NS

    pad_n = E_pad - E
    src = edge_index_list[:, 0, :]
    dst = edge_index_list[:, 1, :]
    pad_src = (jnp.arange(pad_n, dtype=jnp.int32) % N)[None, :] + jnp.zeros((V, 1), jnp.int32)
    pad_dst = (N + jnp.arange(pad_n, dtype=jnp.int32) % (NPAD - N))[None, :] + jnp.zeros((V, 1), jnp.int32)
    src_p = jnp.concatenate([src, pad_src], axis=1)
    dst_p = jnp.concatenate([dst, pad_dst], axis=1)
    voff = (jnp.arange(V, dtype=jnp.int32) * N)[:, None]
    voff_pad = (jnp.arange(V, dtype=jnp.int32) * NPAD)[:, None]
    sidx_g = (src_p + voff).reshape(V, NW, NG, G, CHG)
    didx_r = dst_p.reshape(V, NW, NG, G, CHG)
    didx_deg = (dst_p + voff_pad).reshape(V, NW, CPWD, CH)

    zrow = jnp.zeros((ZR,), f32)
    zblk = jnp.zeros((WZ, D), f32)
    aw = jax.nn.softmax(attn).reshape(V, 1).astype(f32)

    mesh = plsc.VectorSubcoreMesh(core_axis_name="c", subcore_axis_name="s")

    deg_call = pl.kernel(
        functools.partial(_sc_deg_body, V, CPWD),
        out_type=jax.ShapeDtypeStruct((NC * V * NPAD,), f32),
        mesh=mesh,
        scratch_types=[
            pltpu.VMEM((CPWD, CH), jnp.int32),
            pltpu.VMEM((CH,), f32),
            pltpu.VMEM((ZR,), f32),
            pltpu.VMEM_SHARED((V * NPAD,), f32),
        ],
    )
    degp = deg_call(didx_deg, zrow)
    degp4 = degp.reshape(NC, V, NPAD, 1)

    agg_call = pl.kernel(
        functools.partial(_sc_agg_body, V, G),
        out_type=jax.ShapeDtypeStruct((V, NC, NPAD, D), f32),
        mesh=mesh,
        scratch_types=[
            pltpu.VMEM((G, CHG), jnp.int32),
            pltpu.VMEM((G, CHG), jnp.int32),
            pltpu.VMEM((G, CHG), jnp.int32),
            pltpu.VMEM((G, CHG), jnp.int32),
            pltpu.VMEM((CHG, D), f32),
            pltpu.VMEM((CHG, D), f32),
            pltpu.VMEM((CHG, D), f32),
            pltpu.VMEM_SHARED((NPAD, D), f32),
            pltpu.SemaphoreType.DMA,
            pltpu.SemaphoreType.DMA,
            pltpu.SemaphoreType.DMA,
            pltpu.SemaphoreType.DMA,
            pltpu.SemaphoreType.DMA,
            pltpu.SemaphoreType.DMA,
        ],
    )

    stage1 = pl.pallas_call(
        _tc_stage1,
        grid=(NT, V),
        in_specs=[
            pl.BlockSpec((TN, D), lambda t, v: (t, 0)),
            pl.BlockSpec((1, D, D), lambda t, v: (v, 0, 0)),
            pl.BlockSpec((NC, 1, TN, 1), lambda t, v: (0, v, t, 0)),
        ],
        out_specs=[
            pl.BlockSpec((1, TN, D), lambda t, v: (v, t, 0)),
            pl.BlockSpec((1, TN, 1), lambda t, v: (v, t, 0)),
        ],
        out_shape=[
            jax.ShapeDtypeStruct((V, N, D), f32),
            jax.ShapeDtypeStruct((V, N, 1), f32),
        ],
    )
    g1, dinv = stage1(x, W1, degp4)

    acc1 = agg_call(g1.reshape(V * N, D), sidx_g, didx_r, zblk)

    stage2 = pl.pallas_call(
        functools.partial(_tc_stage2, N),
        grid=(V, 2, NT),
        in_specs=[
            pl.BlockSpec((1, NC, TN, D), lambda v, p, t: (v, 0, t, 0)),
            pl.BlockSpec((1, TN, D), lambda v, p, t: (v, t, 0)),
            pl.BlockSpec((1, TN, 1), lambda v, p, t: (v, t, 0)),
            pl.BlockSpec((2, D), lambda v, p, t: (0, 0)),
            pl.BlockSpec((2, D), lambda v, p, t: (0, 0)),
            pl.BlockSpec((1, D, D), lambda v, p, t: (v, 0, 0)),
        ],
        out_specs=pl.BlockSpec((1, TN, D), lambda v, p, t: (v, t, 0)),
        out_shape=jax.ShapeDtypeStruct((V, N, D), f32),
        scratch_shapes=[
            pltpu.VMEM((8, D), f32),
            pltpu.VMEM((8, D), f32),
        ],
    )
    g2 = stage2(acc1, g1, dinv, bn_gamma, bn_beta, W2)

    acc2 = agg_call(g2.reshape(V * N, D), sidx_g, didx_r, zblk)

    stage3 = pl.pallas_call(
        functools.partial(_tc_stage3, N, V),
        grid=(2, NT, V),
        in_specs=[
            pl.BlockSpec((1, NC, TN, D), lambda p, t, v: (v, 0, t, 0)),
            pl.BlockSpec((1, TN, D), lambda p, t, v: (v, t, 0)),
            pl.BlockSpec((1, TN, 1), lambda p, t, v: (v, t, 0)),
            pl.BlockSpec((2, D), lambda p, t, v: (0, 0)),
            pl.BlockSpec((2, D), lambda p, t, v: (0, 0)),
            pl.BlockSpec(memory_space=pltpu.SMEM),
        ],
        out_specs=pl.BlockSpec((TN, D), lambda p, t, v: (t, 0)),
        out_shape=jax.ShapeDtypeStruct((N, D), f32),
        scratch_shapes=[
            pltpu.VMEM((V, 8, D), f32),
            pltpu.VMEM((V, 8, D), f32),
        ],
    )
    return stage3(acc2, g2, dinv, bn_gamma, bn_beta, aw)
